# baseline (device time: 110349 ns/iter reference)
import jax
import jax.numpy as jnp
from jax import lax
from jax.experimental import pallas as pl
from jax.experimental.pallas import tpu as pltpu

N_DEV = 4
V_PER = 16384
N_IDX = 2048
D = 1024
HALF = N_IDX // 2
CHUNK = HALF // N_DEV
SUB = CHUNK // 2
GATHER_SEMS = 16
MESH = pl.DeviceIdType.MESH


def kernel(table, idx):
    my_pos = lax.axis_index("i")
    off = (my_pos * V_PER).astype(jnp.int32)
    local_idx = jnp.clip(idx - off, 0, V_PER - 1).astype(jnp.int32)
    mask2d = ((idx >= off) & (idx < off + V_PER)).astype(jnp.float32)
    mask2d = mask2d.reshape(N_IDX, 1)

    def body(table_ref, lidx_ref, mask_ref, out_ref,
             rs_buf_r, rs_buf_l,
             r_send_sems, r_recv_sems, l_send_sems, l_recv_sems,
             gather_sems):
        p = lax.axis_index("i")
        left = lax.rem(p - 1 + N_DEV, N_DEV)
        right = lax.rem(p + 1, N_DEV)

        def cmod(x):
            return lax.rem(x + 2 * N_DEV, N_DEV)

        def sub_rows(ring, c, u):
            return pl.ds(ring * HALF + c * CHUNK + u * SUB, SUB)

        def chunk_rows(ring, c):
            return pl.ds(ring * HALF + c * CHUNK, CHUNK)

        def gather_issue(slot, base, nrows):
            def blk(b, carry):
                for k in range(GATHER_SEMS):
                    pos = base + b * GATHER_SEMS + k
                    pltpu.make_async_copy(
                        table_ref.at[pl.ds(lidx_ref[pos], 1), :],
                        out_ref.at[pl.ds(pos, 1), :],
                        gather_sems.at[slot],
                    ).start()
                return carry

            lax.fori_loop(0, nrows // GATHER_SEMS, blk, 0)

        def gather_finish(slot, base, nrows):
            pltpu.make_async_copy(
                table_ref.at[pl.ds(0, nrows), :],
                out_ref.at[pl.ds(base, nrows), :],
                gather_sems.at[slot],
            ).wait()
            out_ref[pl.ds(base, nrows), :] = (
                out_ref[pl.ds(base, nrows), :] * mask_ref[pl.ds(base, nrows), :]
            )

        def gather(slot, ring, c):
            base = ring * HALF + c * CHUNK
            gather_issue(slot, base, CHUNK)
            gather_finish(slot, base, CHUNK)

        def rs_desc(ring, s, u, c):
            return pltpu.make_async_remote_copy(
                src_ref=out_ref.at[sub_rows(ring, c, u), :],
                dst_ref=(rs_buf_r if ring == 0 else rs_buf_l).at[
                    s, pl.ds(u * SUB, SUB), :
                ],
                send_sem=(r_send_sems if ring == 0 else l_send_sems).at[
                    2 * s + u
                ],
                recv_sem=(r_recv_sems if ring == 0 else l_recv_sems).at[
                    2 * s + u
                ],
                device_id=((right,) if ring == 0 else (left,)),
                device_id_type=MESH,
            )

        def ag_desc(ring, h, u, c_src, c_dst):
            return pltpu.make_async_remote_copy(
                src_ref=out_ref.at[sub_rows(ring, c_src, u), :],
                dst_ref=out_ref.at[sub_rows(ring, c_dst, u), :],
                send_sem=(r_send_sems if ring == 0 else l_send_sems).at[
                    6 + 2 * h + u
                ],
                recv_sem=(r_recv_sems if ring == 0 else l_recv_sems).at[
                    6 + 2 * h + u
                ],
                device_id=((right,) if ring == 0 else (left,)),
                device_id_type=MESH,
            )

        def rs_send_chunk(ring, s):
            return cmod(p - s) if ring == 0 else cmod(p + s)

        def rs_recv_chunk(ring, s):
            return cmod(p - s - 1) if ring == 0 else cmod(p + s + 1)

        def ag_send_chunk(ring, h):
            return cmod(p + 1 - h) if ring == 0 else cmod(p - 1 + h)

        def ag_recv_chunk(ring, h):
            return cmod(p - h) if ring == 0 else cmod(p + h)

        def sub_base(ring, u):
            return ring * HALF + p * CHUNK + u * SUB

        gather_issue(0, sub_base(0, 0), SUB)
        gather_issue(1, sub_base(1, 0), SUB)

        barrier_sem = pltpu.get_barrier_semaphore()
        for nbr in (left, right):
            pl.semaphore_signal(
                barrier_sem, inc=1, device_id=(nbr,), device_id_type=MESH,
            )
        pl.semaphore_wait(barrier_sem, 2)

        gather_finish(0, sub_base(0, 0), SUB)
        gather_finish(1, sub_base(1, 0), SUB)
        rs_desc(0, 0, 0, p).start()
        rs_desc(1, 0, 0, p).start()

        gather_issue(2, sub_base(0, 1), SUB)
        gather_issue(3, sub_base(1, 1), SUB)
        gather_finish(2, sub_base(0, 1), SUB)
        gather_finish(3, sub_base(1, 1), SUB)
        rs_desc(0, 0, 1, p).start()
        rs_desc(1, 0, 1, p).start()

        gather(4, 0, rs_recv_chunk(0, 0))
        gather(5, 1, rs_recv_chunk(1, 0))

        for s in range(N_DEV - 1):
            for u in (0, 1):
                for ring in (0, 1):
                    rc = rs_recv_chunk(ring, s)
                    rs_desc(ring, s, u, rs_send_chunk(ring, s)).wait_recv()
                    buf = rs_buf_r if ring == 0 else rs_buf_l
                    out_ref[sub_rows(ring, rc, u), :] = (
                        out_ref[sub_rows(ring, rc, u), :]
                        + buf[s, pl.ds(u * SUB, SUB), :]
                    )
                    if s < N_DEV - 2:
                        rs_desc(ring, s + 1, u, rc).start()
                    else:
                        ag_desc(ring, 0, u, rc, rc).start()
            if s < N_DEV - 2:
                gather(6 + 2 * s, 0, rs_recv_chunk(0, s + 1))
                gather(7 + 2 * s, 1, rs_recv_chunk(1, s + 1))

        for h in range(1, N_DEV - 1):
            for u in (0, 1):
                for ring in (0, 1):
                    c_in = ag_recv_chunk(ring, h - 1)
                    ag_desc(
                        ring, h - 1, u, ag_send_chunk(ring, h - 1), c_in
                    ).wait_recv()
                    ag_desc(ring, h, u, c_in, c_in).start()
        for u in (0, 1):
            for ring in (0, 1):
                ag_desc(
                    ring,
                    N_DEV - 2,
                    u,
                    ag_send_chunk(ring, N_DEV - 2),
                    ag_recv_chunk(ring, N_DEV - 2),
                ).wait_recv()

        for ring in (0, 1):
            for s in range(N_DEV - 1):
                for u in (0, 1):
                    rs_desc(ring, s, u, rs_send_chunk(ring, s)).wait_send()
                    ag_desc(
                        ring, s, u, ag_send_chunk(ring, s),
                        ag_send_chunk(ring, s),
                    ).wait_send()

    return pl.pallas_call(
        body,
        out_shape=jax.ShapeDtypeStruct((N_IDX, D), jnp.float32),
        in_specs=[
            pl.BlockSpec(memory_space=pl.ANY),
            pl.BlockSpec(memory_space=pltpu.SMEM),
            pl.BlockSpec(memory_space=pltpu.VMEM),
        ],
        out_specs=pl.BlockSpec(memory_space=pltpu.VMEM),
        scratch_shapes=[
            pltpu.VMEM((N_DEV - 1, CHUNK, D), jnp.float32),
            pltpu.VMEM((N_DEV - 1, CHUNK, D), jnp.float32),
            pltpu.SemaphoreType.DMA((12,)),
            pltpu.SemaphoreType.DMA((12,)),
            pltpu.SemaphoreType.DMA((12,)),
            pltpu.SemaphoreType.DMA((12,)),
            pltpu.SemaphoreType.DMA((10,)),
        ],
        compiler_params=pltpu.CompilerParams(collective_id=0),
    )(table, local_idx, mask2d)


# device time: 84377 ns/iter; 1.3078x vs baseline; 1.3078x over previous
import jax
import jax.numpy as jnp
from jax import lax
from jax.experimental import pallas as pl
from jax.experimental.pallas import tpu as pltpu

N_DEV = 4
V_PER = 16384
N_IDX = 2048
D = 1024
HALF = N_IDX // 2
CHUNK = HALF // N_DEV
SUB = CHUNK // 2
GATHER_SEMS = 16
MESH = pl.DeviceIdType.MESH


def kernel(table, idx):
    my_pos = lax.axis_index("i")
    off = (my_pos * V_PER).astype(jnp.int32)
    owned = (idx >= off) & (idx < off + V_PER)
    local_idx = jnp.where(owned, idx - off, -1).astype(jnp.int32)
    counts16 = owned.reshape(16, SUB).sum(axis=1).astype(jnp.int32)

    def body(table_ref, lidx_ref, cnt16_ref, out_ref,
             rs_buf_r, rs_buf_l,
             r_send_sems, r_recv_sems, l_send_sems, l_recv_sems,
             gather_sems):
        p = lax.axis_index("i")
        left = lax.rem(p - 1 + N_DEV, N_DEV)
        right = lax.rem(p + 1, N_DEV)

        def cmod(x):
            return lax.rem(x + 2 * N_DEV, N_DEV)

        def sub_rows(ring, c, u):
            return pl.ds(ring * HALF + c * CHUNK + u * SUB, SUB)

        def chunk_rows(ring, c):
            return pl.ds(ring * HALF + c * CHUNK, CHUNK)

        def gather_issue(slot, base, nrows):
            def blk(b, carry):
                for k in range(GATHER_SEMS):
                    pos = base + b * GATHER_SEMS + k
                    row = lidx_ref[pos]

                    @pl.when(row >= 0)
                    def _():
                        pltpu.make_async_copy(
                            table_ref.at[pl.ds(row, 1), :],
                            out_ref.at[pl.ds(pos, 1), :],
                            gather_sems.at[slot],
                        ).start()
                return carry

            lax.fori_loop(0, nrows // GATHER_SEMS, blk, 0)

        def gather_finish(slot, cnt):
            def w(n, carry):
                pltpu.make_async_copy(
                    table_ref.at[pl.ds(0, 1), :],
                    out_ref.at[pl.ds(0, 1), :],
                    gather_sems.at[slot],
                ).wait()
                return carry

            lax.fori_loop(0, cnt, w, 0)

        def sub_cnt(ring, c, u):
            return cnt16_ref[ring * 8 + c * 2 + u]

        def chunk_cnt(ring, c):
            return sub_cnt(ring, c, 0) + sub_cnt(ring, c, 1)

        def gather(slot, ring, c):
            base = ring * HALF + c * CHUNK
            gather_issue(slot, base, CHUNK)
            gather_finish(slot, chunk_cnt(ring, c))

        def rs_desc(ring, s, u, c):
            return pltpu.make_async_remote_copy(
                src_ref=out_ref.at[sub_rows(ring, c, u), :],
                dst_ref=(rs_buf_r if ring == 0 else rs_buf_l).at[
                    s, pl.ds(u * SUB, SUB), :
                ],
                send_sem=(r_send_sems if ring == 0 else l_send_sems).at[
                    2 * s + u
                ],
                recv_sem=(r_recv_sems if ring == 0 else l_recv_sems).at[
                    2 * s + u
                ],
                device_id=((right,) if ring == 0 else (left,)),
                device_id_type=MESH,
            )

        def ag_desc(ring, h, u, c_src, c_dst):
            return pltpu.make_async_remote_copy(
                src_ref=out_ref.at[sub_rows(ring, c_src, u), :],
                dst_ref=out_ref.at[sub_rows(ring, c_dst, u), :],
                send_sem=(r_send_sems if ring == 0 else l_send_sems).at[
                    6 + 2 * h + u
                ],
                recv_sem=(r_recv_sems if ring == 0 else l_recv_sems).at[
                    6 + 2 * h + u
                ],
                device_id=((right,) if ring == 0 else (left,)),
                device_id_type=MESH,
            )

        def rs_send_chunk(ring, s):
            return cmod(p - s) if ring == 0 else cmod(p + s)

        def rs_recv_chunk(ring, s):
            return cmod(p - s - 1) if ring == 0 else cmod(p + s + 1)

        def ag_send_chunk(ring, h):
            return cmod(p + 1 - h) if ring == 0 else cmod(p - 1 + h)

        def ag_recv_chunk(ring, h):
            return cmod(p - h) if ring == 0 else cmod(p + h)

        def sub_base(ring, u):
            return ring * HALF + p * CHUNK + u * SUB

        out_ref[...] = jnp.zeros((N_IDX, D), jnp.float32)

        gather_issue(0, sub_base(0, 0), SUB)
        gather_issue(1, sub_base(1, 0), SUB)

        barrier_sem = pltpu.get_barrier_semaphore()
        for nbr in (left, right):
            pl.semaphore_signal(
                barrier_sem, inc=1, device_id=(nbr,), device_id_type=MESH,
            )
        pl.semaphore_wait(barrier_sem, 2)

        gather_finish(0, sub_cnt(0, p, 0))
        gather_finish(1, sub_cnt(1, p, 0))
        rs_desc(0, 0, 0, p).start()
        rs_desc(1, 0, 0, p).start()

        gather_issue(2, sub_base(0, 1), SUB)
        gather_issue(3, sub_base(1, 1), SUB)
        gather_finish(2, sub_cnt(0, p, 1))
        gather_finish(3, sub_cnt(1, p, 1))
        rs_desc(0, 0, 1, p).start()
        rs_desc(1, 0, 1, p).start()

        gather(4, 0, rs_recv_chunk(0, 0))
        gather(5, 1, rs_recv_chunk(1, 0))

        for s in range(N_DEV - 1):
            for u in (0, 1):
                for ring in (0, 1):
                    rc = rs_recv_chunk(ring, s)
                    rs_desc(ring, s, u, rs_send_chunk(ring, s)).wait_recv()
                    buf = rs_buf_r if ring == 0 else rs_buf_l
                    out_ref[sub_rows(ring, rc, u), :] = (
                        out_ref[sub_rows(ring, rc, u), :]
                        + buf[s, pl.ds(u * SUB, SUB), :]
                    )
                    if s < N_DEV - 2:
                        rs_desc(ring, s + 1, u, rc).start()
                    else:
                        ag_desc(ring, 0, u, rc, rc).start()
            if s < N_DEV - 2:
                gather(6 + 2 * s, 0, rs_recv_chunk(0, s + 1))
                gather(7 + 2 * s, 1, rs_recv_chunk(1, s + 1))

        for h in range(1, N_DEV - 1):
            for u in (0, 1):
                for ring in (0, 1):
                    c_in = ag_recv_chunk(ring, h - 1)
                    ag_desc(
                        ring, h - 1, u, ag_send_chunk(ring, h - 1), c_in
                    ).wait_recv()
                    ag_desc(ring, h, u, c_in, c_in).start()
        for u in (0, 1):
            for ring in (0, 1):
                ag_desc(
                    ring,
                    N_DEV - 2,
                    u,
                    ag_send_chunk(ring, N_DEV - 2),
                    ag_recv_chunk(ring, N_DEV - 2),
                ).wait_recv()

        for ring in (0, 1):
            for s in range(N_DEV - 1):
                for u in (0, 1):
                    rs_desc(ring, s, u, rs_send_chunk(ring, s)).wait_send()
                    ag_desc(
                        ring, s, u, ag_send_chunk(ring, s),
                        ag_send_chunk(ring, s),
                    ).wait_send()

    return pl.pallas_call(
        body,
        out_shape=jax.ShapeDtypeStruct((N_IDX, D), jnp.float32),
        in_specs=[
            pl.BlockSpec(memory_space=pl.ANY),
            pl.BlockSpec(memory_space=pltpu.SMEM),
            pl.BlockSpec(memory_space=pltpu.SMEM),
        ],
        out_specs=pl.BlockSpec(memory_space=pltpu.VMEM),
        scratch_shapes=[
            pltpu.VMEM((N_DEV - 1, CHUNK, D), jnp.float32),
            pltpu.VMEM((N_DEV - 1, CHUNK, D), jnp.float32),
            pltpu.SemaphoreType.DMA((12,)),
            pltpu.SemaphoreType.DMA((12,)),
            pltpu.SemaphoreType.DMA((12,)),
            pltpu.SemaphoreType.DMA((12,)),
            pltpu.SemaphoreType.DMA((10,)),
        ],
        compiler_params=pltpu.CompilerParams(collective_id=0),
    )(table, local_idx, counts16)
